# baseline (speedup 1.0000x reference)
import functools

import jax
import jax.numpy as jnp
from jax import lax
from jax.experimental import pallas as pl
from jax.experimental.pallas import tpu as pltpu
from jax.experimental.pallas import tpu_sc as plsc

N = 10000
E = 320000
D = 128

NC = 2
NS = 16
NW = NC * NS
CHUNK = 128
NCHUNK = 80
GCH = 8
NGRP = NCHUNK // GCH
EPW = CHUNK * NCHUNK
EPAD = EPW * NW
NCHUNK0 = 160
NCHUNK1 = 2 * NCHUNK - NCHUNK0
NGRP0 = NCHUNK0 // GCH
NGRP1 = NCHUNK1 // GCH
NPAD = 10112
RPS = NPAD // NS
DW = 16

_BLK = (0, 128, 256, 384, 504)

_MESH = plsc.VectorSubcoreMesh(core_axis_name="c", subcore_axis_name="s")


def _make_sc_agg(W):
    @functools.partial(
        pl.kernel,
        mesh=_MESH,
        out_type=jax.ShapeDtypeStruct((NC * NPAD, W), jnp.float32),
        scratch_types=[
            pltpu.VMEM((GCH, CHUNK), jnp.int32),
            pltpu.VMEM((GCH, CHUNK), jnp.int32),
            pltpu.VMEM((CHUNK, W), jnp.float32),
            pltpu.VMEM((CHUNK, W), jnp.float32),
            pltpu.VMEM_SHARED((NPAD, W), jnp.float32),
            pltpu.SemaphoreType.DMA,
            pltpu.SemaphoreType.DMA,
        ],
    )
    def agg(h_hbm, srcp, dstp, zrow, agg_out,
            src_v, dst_v, rows_a, rows_b, acc_sh, sem_a, sem_b):
        c = lax.axis_index("c")
        s = lax.axis_index("s")
        wid = c * NS + s
        bufs = ((rows_a, sem_a), (rows_b, sem_b))

        row0 = s * RPS
        pltpu.sync_copy(zrow, rows_a)
        for off in _BLK:
            pltpu.sync_copy(rows_a, acc_sh.at[pl.ds(row0 + off, CHUNK)])
        plsc.subcore_barrier()

        base_chunk = c * (NS * NCHUNK0) + s * jnp.where(c == 0, NCHUNK0, NCHUNK1)
        ngrp = jnp.where(c == 0, NGRP0, NGRP1)

        def group(g, carry):
            base = base_chunk + g * GCH
            pltpu.sync_copy(srcp.at[pl.ds(base, GCH)], src_v)
            pltpu.sync_copy(dstp.at[pl.ds(base, GCH)], dst_v)
            cp = pltpu.async_copy(h_hbm.at[src_v.at[0]], rows_a, sem_a)
            for b in range(GCH):
                buf, _ = bufs[b % 2]
                cp.wait()
                if b < GCH - 1:
                    nbuf, nsem = bufs[(b + 1) % 2]
                    cp = pltpu.async_copy(h_hbm.at[src_v.at[b + 1]], nbuf, nsem)
                pltpu.sync_copy(buf, acc_sh.at[dst_v.at[b]], add=True)
            return carry

        lax.fori_loop(0, ngrp, group, 0)
        plsc.subcore_barrier()

        out0 = wid * RPS
        for off in _BLK:
            pltpu.sync_copy(acc_sh.at[pl.ds(row0 + off, CHUNK)], rows_a)
            pltpu.sync_copy(rows_a, agg_out.at[pl.ds(out0 + off, CHUNK)])

    return agg


_sc_agg = _make_sc_agg(D)


@functools.partial(
    pl.kernel,
    mesh=_MESH,
    out_type=jax.ShapeDtypeStruct((NC * NPAD, D), jnp.float32),
    scratch_types=[
        pltpu.VMEM((GCH, CHUNK), jnp.int32),
        pltpu.VMEM((CHUNK, D), jnp.float32),
        pltpu.VMEM_SHARED((NPAD, D), jnp.float32),
    ],
)
def _sc_deg(onesrow, dstp, zrow, deg_out, dst_v, rows_v, acc_sh):
    c = lax.axis_index("c")
    s = lax.axis_index("s")
    wid = c * NS + s

    row0 = s * RPS
    pltpu.sync_copy(zrow, rows_v)
    for off in _BLK:
        pltpu.sync_copy(rows_v, acc_sh.at[pl.ds(row0 + off, CHUNK)])
    plsc.subcore_barrier()

    pltpu.sync_copy(onesrow, rows_v)

    def group(g, carry):
        base = wid * NCHUNK + g * GCH
        pltpu.sync_copy(dstp.at[pl.ds(base, GCH)], dst_v)
        for b in range(GCH):
            pltpu.sync_copy(rows_v, acc_sh.at[dst_v.at[b]], add=True)
        return carry

    lax.fori_loop(0, NGRP, group, 0)
    plsc.subcore_barrier()

    out0 = wid * RPS
    for off in _BLK:
        pltpu.sync_copy(acc_sh.at[pl.ds(row0 + off, CHUNK)], rows_v)
        pltpu.sync_copy(rows_v, deg_out.at[pl.ds(out0 + off, CHUNK)])


RB = 400
GRID = N // RB


def _tc_combine_body(aggp, degp, h, wl, b, wr, o):
    agg = aggp[0] + aggp[1]
    deg = degp[0][:, 0:1] + degp[1][:, 0:1]
    mean = agg * (1.0 / jnp.maximum(deg, 1.0))
    y = (lax.dot_general(mean, wl[...], (((1,), (1,)), ((), ())),
                         preferred_element_type=jnp.float32)
         + lax.dot_general(h[...], wr[...], (((1,), (1,)), ((), ())),
                           preferred_element_type=jnp.float32)
         + b[...])
    o[...] = jnp.maximum(y, 0.0)


def _tc_combine_fc_body(aggp, degp, h, wl, b, wr, wfc, bfc, o):
    agg = aggp[0] + aggp[1]
    deg = degp[0][:, 0:1] + degp[1][:, 0:1]
    mean = agg * (1.0 / jnp.maximum(deg, 1.0))
    y = (lax.dot_general(mean, wl[...], (((1,), (1,)), ((), ())),
                         preferred_element_type=jnp.float32)
         + lax.dot_general(h[...], wr[...], (((1,), (1,)), ((), ())),
                           preferred_element_type=jnp.float32)
         + b[...])
    y = jnp.maximum(y, 0.0)
    o[...] = lax.dot_general(y, wfc[...], (((1,), (1,)), ((), ())),
                             preferred_element_type=jnp.float32) + bfc[...]


_AGG_SPEC = pl.BlockSpec((2, RB, D), lambda i: (0, i, 0))
_DEG_SPEC = pl.BlockSpec((2, RB, DW), lambda i: (0, i, 0))
_ROW_SPEC = pl.BlockSpec((RB, D), lambda i: (i, 0))
_W_SPEC = pl.BlockSpec((D, D), lambda i: (0, 0))
_B_SPEC = pl.BlockSpec((1, D), lambda i: (0, 0))

_tc_combine = pl.pallas_call(
    _tc_combine_body,
    grid=(GRID,),
    in_specs=[_AGG_SPEC, _DEG_SPEC, _ROW_SPEC, _W_SPEC, _B_SPEC, _W_SPEC],
    out_specs=_ROW_SPEC,
    out_shape=jax.ShapeDtypeStruct((N, D), jnp.float32),
)

_tc_combine_fc = pl.pallas_call(
    _tc_combine_fc_body,
    grid=(GRID,),
    in_specs=[_AGG_SPEC, _DEG_SPEC, _ROW_SPEC, _W_SPEC, _B_SPEC, _W_SPEC,
              _W_SPEC, _B_SPEC],
    out_specs=_ROW_SPEC,
    out_shape=jax.ShapeDtypeStruct((N, D), jnp.float32),
)


def kernel(x, edge_index, W1l, b1, W1r, W2l, b2, W2r, W3l, b3, W3r, Wfc, bfc):
    src = edge_index[0].astype(jnp.int32)
    dst = edge_index[1].astype(jnp.int32)
    npad = EPAD - E
    srcp = jnp.concatenate([src, jnp.zeros((npad,), jnp.int32)]
                           ).reshape(NW * NCHUNK, CHUNK)
    dstp = jnp.concatenate([dst, jnp.full((npad,), N, jnp.int32)]
                           ).reshape(NW * NCHUNK, CHUNK)
    zrow = jnp.zeros((CHUNK, D), jnp.float32)
    onesrow = jnp.ones((CHUNK, D), jnp.float32)

    degp = _sc_deg(onesrow, dstp, zrow).reshape(NC, NPAD, D)[:, :, :DW]
    aggp1 = _sc_agg(x, srcp, dstp, zrow).reshape(NC, NPAD, D)
    h1 = _tc_combine(aggp1, degp, x, W1l, b1.reshape(1, D), W1r)
    aggp2 = _sc_agg(h1, srcp, dstp, zrow).reshape(NC, NPAD, D)
    h2 = _tc_combine(aggp2, degp, h1, W2l, b2.reshape(1, D), W2r)
    aggp3 = _sc_agg(h2, srcp, dstp, zrow).reshape(NC, NPAD, D)
    out = _tc_combine_fc(aggp3, degp, h2, W3l, b3.reshape(1, D), W3r,
                         Wfc, bfc.reshape(1, D))
    return out


# speedup vs baseline: 1.4824x; 1.4824x over previous
import functools

import jax
import jax.numpy as jnp
from jax import lax
from jax.experimental import pallas as pl
from jax.experimental.pallas import tpu as pltpu
from jax.experimental.pallas import tpu_sc as plsc

N = 10000
E = 320000
D = 128

NC = 2
NS = 16
NW = NC * NS
CHUNK = 128
NCHUNK = 80
GCH = 8
NGRP = NCHUNK // GCH
EPW = CHUNK * NCHUNK
EPAD = EPW * NW
NCHUNK0 = 144
NCHUNK1 = 2 * NCHUNK - NCHUNK0
NGRP0 = NCHUNK0 // GCH
NGRP1 = NCHUNK1 // GCH
NPAD = 10112
RPS = NPAD // NS
DW = 16

_BLK = (0, 128, 256, 384, 504)

_MESH = plsc.VectorSubcoreMesh(core_axis_name="c", subcore_axis_name="s")


def _make_sc_agg(W):
    @functools.partial(
        pl.kernel,
        mesh=_MESH,
        out_type=jax.ShapeDtypeStruct((NC * NPAD, W), jnp.float32),
        scratch_types=[
            pltpu.VMEM((GCH, CHUNK), jnp.int32),
            pltpu.VMEM((GCH, CHUNK), jnp.int32),
            pltpu.VMEM((CHUNK, W), jnp.float32),
            pltpu.VMEM((CHUNK, W), jnp.float32),
            pltpu.VMEM_SHARED((NPAD, W), jnp.float32),
            pltpu.SemaphoreType.DMA,
            pltpu.SemaphoreType.DMA,
        ],
    )
    def agg(h_hbm, srcp, dstp, zrow, agg_out,
            src_v, dst_v, rows_a, rows_b, acc_sh, sem_a, sem_b):
        c = lax.axis_index("c")
        s = lax.axis_index("s")
        wid = c * NS + s
        bufs = ((rows_a, sem_a), (rows_b, sem_b))

        row0 = s * RPS
        pltpu.sync_copy(zrow, rows_a)
        for off in _BLK:
            pltpu.sync_copy(rows_a, acc_sh.at[pl.ds(row0 + off, CHUNK)])
        plsc.subcore_barrier()

        base_chunk = c * (NS * NCHUNK0) + s * jnp.where(c == 0, NCHUNK0, NCHUNK1)
        ngrp = jnp.where(c == 0, NGRP0, NGRP1)

        def group(g, carry):
            base = base_chunk + g * GCH
            pltpu.sync_copy(srcp.at[pl.ds(base, GCH)], src_v)
            pltpu.sync_copy(dstp.at[pl.ds(base, GCH)], dst_v)
            cp = pltpu.async_copy(h_hbm.at[src_v.at[0]], rows_a, sem_a)
            for b in range(GCH):
                buf, _ = bufs[b % 2]
                cp.wait()
                if b < GCH - 1:
                    nbuf, nsem = bufs[(b + 1) % 2]
                    cp = pltpu.async_copy(h_hbm.at[src_v.at[b + 1]], nbuf, nsem)
                pltpu.sync_copy(buf, acc_sh.at[dst_v.at[b]], add=True)
            return carry

        lax.fori_loop(0, ngrp, group, 0)
        plsc.subcore_barrier()

        out0 = wid * RPS
        for off in _BLK:
            pltpu.sync_copy(acc_sh.at[pl.ds(row0 + off, CHUNK)], rows_a)
            pltpu.sync_copy(rows_a, agg_out.at[pl.ds(out0 + off, CHUNK)])

    return agg


_sc_agg = _make_sc_agg(D)


@functools.partial(
    pl.kernel,
    mesh=_MESH,
    out_type=jax.ShapeDtypeStruct((NC * NPAD, D), jnp.float32),
    scratch_types=[
        pltpu.VMEM((GCH, CHUNK), jnp.int32),
        pltpu.VMEM((CHUNK, D), jnp.float32),
        pltpu.VMEM_SHARED((NPAD, D), jnp.float32),
    ],
)
def _sc_deg(onesrow, dstp, zrow, deg_out, dst_v, rows_v, acc_sh):
    c = lax.axis_index("c")
    s = lax.axis_index("s")
    wid = c * NS + s

    row0 = s * RPS
    pltpu.sync_copy(zrow, rows_v)
    for off in _BLK:
        pltpu.sync_copy(rows_v, acc_sh.at[pl.ds(row0 + off, CHUNK)])
    plsc.subcore_barrier()

    pltpu.sync_copy(onesrow, rows_v)

    def group(g, carry):
        base = wid * NCHUNK + g * GCH
        pltpu.sync_copy(dstp.at[pl.ds(base, GCH)], dst_v)
        for b in range(GCH):
            pltpu.sync_copy(rows_v, acc_sh.at[dst_v.at[b]], add=True)
        return carry

    lax.fori_loop(0, NGRP, group, 0)
    plsc.subcore_barrier()

    out0 = wid * RPS
    for off in _BLK:
        pltpu.sync_copy(acc_sh.at[pl.ds(row0 + off, CHUNK)], rows_v)
        pltpu.sync_copy(rows_v, deg_out.at[pl.ds(out0 + off, CHUNK)])


RB = 400
GRID = N // RB


def _tc_combine_body(aggp, degp, h, wl, b, wr, o):
    agg = aggp[0] + aggp[1]
    deg = degp[0][:, 0:1] + degp[1][:, 0:1]
    mean = agg * (1.0 / jnp.maximum(deg, 1.0))
    y = (lax.dot_general(mean, wl[...], (((1,), (1,)), ((), ())),
                         preferred_element_type=jnp.float32)
         + lax.dot_general(h[...], wr[...], (((1,), (1,)), ((), ())),
                           preferred_element_type=jnp.float32)
         + b[...])
    o[...] = jnp.maximum(y, 0.0)


def _tc_combine_fc_body(aggp, degp, h, wl, b, wr, wfc, bfc, o):
    agg = aggp[0] + aggp[1]
    deg = degp[0][:, 0:1] + degp[1][:, 0:1]
    mean = agg * (1.0 / jnp.maximum(deg, 1.0))
    y = (lax.dot_general(mean, wl[...], (((1,), (1,)), ((), ())),
                         preferred_element_type=jnp.float32)
         + lax.dot_general(h[...], wr[...], (((1,), (1,)), ((), ())),
                           preferred_element_type=jnp.float32)
         + b[...])
    y = jnp.maximum(y, 0.0)
    o[...] = lax.dot_general(y, wfc[...], (((1,), (1,)), ((), ())),
                             preferred_element_type=jnp.float32) + bfc[...]


_AGG_SPEC = pl.BlockSpec((2, RB, D), lambda i: (0, i, 0))
_DEG_SPEC = pl.BlockSpec((2, RB, DW), lambda i: (0, i, 0))
_ROW_SPEC = pl.BlockSpec((RB, D), lambda i: (i, 0))
_W_SPEC = pl.BlockSpec((D, D), lambda i: (0, 0))
_B_SPEC = pl.BlockSpec((1, D), lambda i: (0, 0))

_tc_combine = pl.pallas_call(
    _tc_combine_body,
    grid=(GRID,),
    in_specs=[_AGG_SPEC, _DEG_SPEC, _ROW_SPEC, _W_SPEC, _B_SPEC, _W_SPEC],
    out_specs=_ROW_SPEC,
    out_shape=jax.ShapeDtypeStruct((N, D), jnp.float32),
)

_tc_combine_fc = pl.pallas_call(
    _tc_combine_fc_body,
    grid=(GRID,),
    in_specs=[_AGG_SPEC, _DEG_SPEC, _ROW_SPEC, _W_SPEC, _B_SPEC, _W_SPEC,
              _W_SPEC, _B_SPEC],
    out_specs=_ROW_SPEC,
    out_shape=jax.ShapeDtypeStruct((N, D), jnp.float32),
)


def kernel(x, edge_index, W1l, b1, W1r, W2l, b2, W2r, W3l, b3, W3r, Wfc, bfc):
    src = edge_index[0].astype(jnp.int32)
    dst = edge_index[1].astype(jnp.int32)
    npad = EPAD - E
    srcp = jnp.concatenate([src, jnp.zeros((npad,), jnp.int32)]
                           ).reshape(NW * NCHUNK, CHUNK)
    dstp = jnp.concatenate([dst, jnp.full((npad,), N, jnp.int32)]
                           ).reshape(NW * NCHUNK, CHUNK)
    zrow = jnp.zeros((CHUNK, D), jnp.float32)
    onesrow = jnp.ones((CHUNK, D), jnp.float32)

    degp = _sc_deg(onesrow, dstp, zrow).reshape(NC, NPAD, D)[:, :, :DW]
    aggp1 = _sc_agg(x, srcp, dstp, zrow).reshape(NC, NPAD, D)
    h1 = _tc_combine(aggp1, degp, x, W1l, b1.reshape(1, D), W1r)
    aggp2 = _sc_agg(h1, srcp, dstp, zrow).reshape(NC, NPAD, D)
    h2 = _tc_combine(aggp2, degp, h1, W2l, b2.reshape(1, D), W2r)
    aggp3 = _sc_agg(h2, srcp, dstp, zrow).reshape(NC, NPAD, D)
    out = _tc_combine_fc(aggp3, degp, h2, W3l, b3.reshape(1, D), W3r,
                         Wfc, bfc.reshape(1, D))
    return out
